# Initial kernel scaffold; baseline (speedup 1.0000x reference)
import functools

import jax
import jax.numpy as jnp
from jax import lax
from jax.experimental import pallas as pl
from jax.experimental.pallas import tpu as pltpu
from jax.experimental.pallas import tpu_sc as plsc

B, P, D, K = 2, 4096, 768, 16
NC, NS, L = 2, 16, 16
NW = NC * NS
ROWS = B * P
RPW = ROWS // NW
G = 4
NBUF = 2
NG = RPW // G
DV = D // L
DU = 4

_mesh = plsc.VectorSubcoreMesh(
    core_axis_name="c", subcore_axis_name="s", num_cores=NC, num_subcores=NS
)


@functools.partial(
    pl.kernel,
    out_type=jax.ShapeDtypeStruct((ROWS, D), jnp.float32),
    mesh=_mesh,
    scratch_types=[
        [pltpu.VMEM((G * K,), jnp.int32) for _ in range(NBUF)],
        [pltpu.VMEM((G * K,), jnp.float32) for _ in range(NBUF)],
        [pltpu.VMEM((G * K, D), jnp.float32) for _ in range(NBUF)],
        pltpu.VMEM((G, D), jnp.float32),
        [pltpu.SemaphoreType.DMA for _ in range(NBUF)],
    ],
)
def _wormhole_gather(x_hbm, routes_hbm, weights_hbm, out_hbm,
                     idx_bufs, w_bufs, row_bufs, out_v, sems):
    wid = lax.axis_index("s") * NC + lax.axis_index("c")
    wstart = wid * RPW
    boff = (wid // (NW // B)) * P
    off_splat = jnp.broadcast_to(boff, (L,)).astype(jnp.int32)

    def issue(c, slot):
        base_k = pl.multiple_of((wstart + c * G) * K, G * K)
        pltpu.sync_copy(routes_hbm.at[pl.ds(base_k, G * K)], idx_bufs[slot])
        for i in range(G * K // L):
            sl = pl.ds(i * L, L)
            idx_bufs[slot][sl] = idx_bufs[slot][sl] + off_splat
        pltpu.sync_copy(weights_hbm.at[pl.ds(base_k, G * K)], w_bufs[slot])
        pltpu.make_async_copy(
            x_hbm.at[idx_bufs[slot]], row_bufs[slot], sems[slot]
        ).start()

    def compute(c, slot):
        base = wstart + c * G
        pltpu.make_async_copy(
            x_hbm.at[idx_bufs[slot]], row_bufs[slot], sems[slot]
        ).wait()
        rows = row_bufs[slot]
        for g in range(G):
            ws = [
                plsc.load_gather(
                    w_bufs[slot], [jnp.full((L,), g * K + k, jnp.int32)]
                )
                for k in range(K)
            ]

            def dbody(d, carry, g=g, ws=ws):
                for u in range(DU):
                    sl = pl.ds((d * DU + u) * L, L)
                    acc = rows[g * K, sl] * ws[0]
                    for k in range(1, K):
                        acc = acc + rows[g * K + k, sl] * ws[k]
                    out_v[g, sl] = acc
                return carry

            lax.fori_loop(0, DV // DU, dbody, 0)
        pltpu.sync_copy(out_v, out_hbm.at[pl.ds(base, G)])

    issue(0, 0)

    def outer(cc, carry):
        for bslot in range(NBUF):
            c = cc * NBUF + bslot
            nxt_slot = (bslot + 1) % NBUF

            @pl.when(c + 1 < NG)
            def _issue_next():
                issue(c + 1, nxt_slot)

            compute(c, bslot)
        return carry

    lax.fori_loop(0, NG // NBUF, outer, 0)


def kernel(x, routes, weights):
    x_flat = x.reshape(ROWS, D)
    r_flat = routes.astype(jnp.int32).reshape(ROWS * K)
    w_flat = weights.reshape(ROWS * K)
    out = _wormhole_gather(x_flat, r_flat, w_flat)
    return out.reshape(B, P, D)


# speedup vs baseline: 8.3928x; 8.3928x over previous
import functools

import jax
import jax.numpy as jnp
from jax import lax
from jax.experimental import pallas as pl
from jax.experimental.pallas import tpu as pltpu
from jax.experimental.pallas import tpu_sc as plsc

B, P, D, K = 2, 4096, 768, 16
NC, NS, L = 2, 16, 16
NW = NC * NS
ROWS = B * P
RPW = ROWS // NW
G = 4
NBUF = 2
NG = RPW // G
DV = D // L
DU = 4

_mesh = plsc.VectorSubcoreMesh(
    core_axis_name="c", subcore_axis_name="s", num_cores=NC, num_subcores=NS
)


@functools.partial(
    pl.kernel,
    out_type=jax.ShapeDtypeStruct((ROWS, D), jnp.float32),
    mesh=_mesh,
    scratch_types=[
        [pltpu.VMEM((G * K,), jnp.int32) for _ in range(NBUF)],
        [pltpu.VMEM((G * K,), jnp.float32) for _ in range(NBUF)],
        [pltpu.VMEM((G * K, D), jnp.float32) for _ in range(NBUF)],
        pltpu.VMEM((G, D), jnp.float32),
        [pltpu.SemaphoreType.DMA for _ in range(NBUF)],
    ],
)
def _wormhole_gather(x_hbm, routes_hbm, weights_hbm, out_hbm,
                     idx_bufs, w_bufs, row_bufs, out_v, sems):
    wid = lax.axis_index("s") * NC + lax.axis_index("c")
    wstart = wid * RPW
    boff = (wid // (NW // B)) * P
    off_splat = jnp.broadcast_to(boff, (L,)).astype(jnp.int32)

    def issue(c, slot):
        base_k = pl.multiple_of((wstart + c * G) * K, G * K)
        pltpu.sync_copy(routes_hbm.at[pl.ds(base_k, G * K)], idx_bufs[slot])
        for i in range(G * K // L):
            sl = pl.ds(i * L, L)
            idx_bufs[slot][sl] = idx_bufs[slot][sl] + off_splat
        pltpu.sync_copy(weights_hbm.at[pl.ds(base_k, G * K)], w_bufs[slot])
        pltpu.make_async_copy(
            x_hbm.at[idx_bufs[slot]], row_bufs[slot], sems[slot]
        ).start()

    def compute(c, slot):
        base = wstart + c * G
        pltpu.make_async_copy(
            x_hbm.at[idx_bufs[slot]], row_bufs[slot], sems[slot]
        ).wait()
        rows = row_bufs[slot]
        for g in range(G):
            wv = w_bufs[slot][pl.ds(g * K, K)]
            ws = [jnp.broadcast_to(wv[k], (L,)) for k in range(K)]

            def dbody(d, carry, g=g, ws=ws):
                for u in range(DU):
                    sl = pl.ds((d * DU + u) * L, L)
                    acc = rows[g * K, sl] * ws[0]
                    for k in range(1, K):
                        acc = acc + rows[g * K + k, sl] * ws[k]
                    out_v[g, sl] = acc
                return carry

            lax.fori_loop(0, DV // DU, dbody, 0)
        pltpu.sync_copy(out_v, out_hbm.at[pl.ds(base, G)])

    issue(0, 0)

    def outer(cc, carry):
        for bslot in range(NBUF):
            c = cc * NBUF + bslot
            nxt_slot = (bslot + 1) % NBUF

            @pl.when(c + 1 < NG)
            def _issue_next():
                issue(c + 1, nxt_slot)

            compute(c, bslot)
        return carry

    lax.fori_loop(0, NG // NBUF, outer, 0)


def kernel(x, routes, weights):
    x_flat = x.reshape(ROWS, D)
    r_flat = routes.astype(jnp.int32).reshape(ROWS * K)
    w_flat = weights.reshape(ROWS * K)
    out = _wormhole_gather(x_flat, r_flat, w_flat)
    return out.reshape(B, P, D)


# speedup vs baseline: 10.5508x; 1.2571x over previous
import functools

import jax
import jax.numpy as jnp
from jax import lax
from jax.experimental import pallas as pl
from jax.experimental.pallas import tpu as pltpu
from jax.experimental.pallas import tpu_sc as plsc

B, P, D, K = 2, 4096, 768, 16
NC, NS, L = 2, 16, 16
NW = NC * NS
ROWS = B * P
RPW = ROWS // NW
G = 4
NBUF = 2
NG = RPW // G
DV = D // L
DU = 4

_mesh = plsc.VectorSubcoreMesh(
    core_axis_name="c", subcore_axis_name="s", num_cores=NC, num_subcores=NS
)


@functools.partial(
    pl.kernel,
    out_type=jax.ShapeDtypeStruct((ROWS, D), jnp.float32),
    mesh=_mesh,
    scratch_types=[
        [pltpu.VMEM((G * K,), jnp.int32) for _ in range(NBUF)],
        [pltpu.VMEM((G * K,), jnp.float32) for _ in range(NBUF)],
        [pltpu.VMEM((G * K, D), jnp.float32) for _ in range(NBUF)],
        pltpu.VMEM((G, D), jnp.float32),
        [pltpu.SemaphoreType.DMA for _ in range(NBUF)],
    ],
)
def _wormhole_gather(x_hbm, routes_hbm, weights_hbm, out_hbm,
                     idx_bufs, w_bufs, row_bufs, out_v, sems):
    wid = lax.axis_index("s") * NC + lax.axis_index("c")
    wstart = wid * RPW
    boff = (wid // (NW // B)) * P
    off_splat = jnp.broadcast_to(boff, (L,)).astype(jnp.int32)

    def issue(c, slot):
        base_k = pl.multiple_of((wstart + c * G) * K, G * K)
        pltpu.sync_copy(routes_hbm.at[pl.ds(base_k, G * K)], idx_bufs[slot])
        for i in range(G * K // L):
            sl = pl.ds(i * L, L)
            idx_bufs[slot][sl] = idx_bufs[slot][sl] + off_splat
        pltpu.sync_copy(weights_hbm.at[pl.ds(base_k, G * K)], w_bufs[slot])
        pltpu.make_async_copy(
            x_hbm.at[idx_bufs[slot]], row_bufs[slot], sems[slot]
        ).start()

    def compute(c, slot):
        base = wstart + c * G
        pltpu.make_async_copy(
            x_hbm.at[idx_bufs[slot]], row_bufs[slot], sems[slot]
        ).wait()
        rows = row_bufs[slot]
        for g in range(G):
            wv = w_bufs[slot][pl.ds(g * K, K)]
            ws = [jnp.broadcast_to(wv[k], (L,)) for k in range(K)]

            def dbody(d, carry, g=g, ws=ws):
                for u in range(DU):
                    sl = pl.ds((d * DU + u) * L, L)
                    acc = [rows[g * K + a, sl] * ws[a] for a in range(4)]
                    for k in range(4, K):
                        acc[k % 4] = acc[k % 4] + rows[g * K + k, sl] * ws[k]
                    out_v[g, sl] = (acc[0] + acc[1]) + (acc[2] + acc[3])
                return carry

            lax.fori_loop(0, DV // DU, dbody, 0)
        pltpu.sync_copy(out_v, out_hbm.at[pl.ds(base, G)])

    issue(0, 0)

    def outer(cc, carry):
        for bslot in range(NBUF):
            c = cc * NBUF + bslot
            nxt_slot = (bslot + 1) % NBUF

            @pl.when(c + 1 < NG)
            def _issue_next():
                issue(c + 1, nxt_slot)

            compute(c, bslot)
        return carry

    lax.fori_loop(0, NG // NBUF, outer, 0)


def kernel(x, routes, weights):
    x_flat = x.reshape(ROWS, D)
    r_flat = routes.astype(jnp.int32).reshape(ROWS * K)
    w_flat = weights.reshape(ROWS * K)
    out = _wormhole_gather(x_flat, r_flat, w_flat)
    return out.reshape(B, P, D)
